# baseline (speedup 1.0000x reference)
import jax
import jax.numpy as jnp
from jax import lax
from jax.experimental import pallas as pl
from jax.experimental.pallas import tpu as pltpu
from jax.experimental.pallas import tpu_sc as plsc

_N = 10000
_E = 320000
_G = 64
_D = 128
_DH = 64
_DV = 64
_NSW = 2


_NT = 32
_TPT = _E // _NT
_C = 80
_NCH = _TPT // _C
_ZR = 40
_NZC = _N // _ZR

_ROWS_BLK = 400
_NBLK = _N // _ROWS_BLK


def _gelu(x):
    return 0.5 * x * (1.0 + lax.erf(x * 0.7071067811865476))


def _mm(a, b):
    return lax.dot_general(a, b, (((1,), (1,)), ((), ())),
                           preferred_element_type=jnp.float32)


def _proj_body(x_ref, m_ref, inv_ref, wk, bk, wq, bq, wv, bv, wr, br,
               kqv_out, r_out):
    xb = x_ref[...]
    m = m_ref[...]
    inv = inv_ref[0, 0]
    mu = jnp.sum(xb * m, axis=-1, keepdims=True) * inv
    xc = (xb - mu) * m
    var = jnp.sum(xc * xc, axis=-1, keepdims=True) * inv
    xn = xc * lax.rsqrt(var + 1e-5)
    k = _mm(xn, wk[...]) + bk[...]
    q = _mm(xn, wq[...]) + bq[...]
    v = _mm(xn, wv[...]) + bv[...]
    kqv_out[...] = jnp.concatenate([k, q, v], axis=1)
    r_out[...] = _mm(xb, wr[...]) + br[...]


def _stage_proj(x, msk, invd, wk, bk, wq, bq, wv, bv, wr, br):
    wspec = pl.BlockSpec((_D, _D), lambda i: (0, 0))
    bspec = pl.BlockSpec((1, _D), lambda i: (0, 0))
    return pl.pallas_call(
        _proj_body,
        grid=(_NBLK,),
        in_specs=[pl.BlockSpec((_ROWS_BLK, _D), lambda i: (i, 0)),
                  bspec, pl.BlockSpec((1, 1), lambda i: (0, 0)),
                  wspec, bspec, wspec, bspec, wspec, bspec, wspec, bspec],
        out_specs=[pl.BlockSpec((_ROWS_BLK, 3 * _D), lambda i: (i, 0)),
                   pl.BlockSpec((_ROWS_BLK, _D), lambda i: (i, 0))],
        out_shape=[jax.ShapeDtypeStruct((_N, 3 * _D), jnp.float32),
                   jax.ShapeDtypeStruct((_N, _D), jnp.float32)],
    )(x, msk, invd, wk, bk, wq, bq, wv, bv, wr, br)


def _post_body(ab_ref, den_ref, r_ref, h_out):
    s = ab_ref[0] + ab_ref[1]
    num = s[:, :_D]
    den = den_ref[...]
    aggr = num / (den + 1e-16)
    h_out[...] = _gelu(aggr) - r_ref[...]


def _stage_post(acc_abc, den, r):
    return pl.pallas_call(
        _post_body,
        grid=(_NBLK,),
        in_specs=[pl.BlockSpec((2, _ROWS_BLK, 2 * _DV), lambda i: (0, i, 0)),
                  pl.BlockSpec((_ROWS_BLK, 1), lambda i: (i, 0)),
                  pl.BlockSpec((_ROWS_BLK, _D), lambda i: (i, 0))],
        out_specs=pl.BlockSpec((_ROWS_BLK, _D), lambda i: (i, 0)),
        out_shape=jax.ShapeDtypeStruct((_N, _D), jnp.float32),
    )(acc_abc, den, r)


def _pool_body(h_ref, b_ref, pooled_ref, counts_ref):
    i = pl.program_id(0)
    h = h_ref[...]
    bvec = b_ref[0]
    oh = (lax.broadcasted_iota(jnp.int32, (_G, _ROWS_BLK), 0)
          == jnp.broadcast_to(bvec, (_G, _ROWS_BLK))).astype(jnp.float32)
    contrib = lax.dot_general(oh, h, (((1,), (0,)), ((), ())),
                              preferred_element_type=jnp.float32)
    cnt = jnp.broadcast_to(jnp.sum(oh, axis=1, keepdims=True), (_G, _D))

    @pl.when(i == 0)
    def _():
        pooled_ref[...] = contrib
        counts_ref[...] = cnt

    @pl.when(i > 0)
    def _():
        pooled_ref[...] += contrib
        counts_ref[...] += cnt

    @pl.when(i == _NBLK - 1)
    def _():
        pooled_ref[...] = pooled_ref[...] / jnp.maximum(counts_ref[...], 1.0)


def _stage_pool(h, batch):
    batch3 = batch.astype(jnp.int32).reshape(_NBLK, 1, _ROWS_BLK)
    gspec = pl.BlockSpec((_G, _D), lambda i: (0, 0))
    pooled, _ = pl.pallas_call(
        _pool_body,
        grid=(_NBLK,),
        in_specs=[pl.BlockSpec((_ROWS_BLK, _D), lambda i: (i, 0)),
                  pl.BlockSpec((1, 1, _ROWS_BLK), lambda i: (i, 0, 0))],
        out_specs=[gspec, gspec],
        out_shape=[jax.ShapeDtypeStruct((_G, _D), jnp.float32),
                   jax.ShapeDtypeStruct((_G, _D), jnp.float32)],
    )(h, batch3)
    return pooled


def _edge_body(k_hbm, q_hbm, va_hbm, vb_hbm, src_hbm,
               dst_hbm, ea_hbm, outa_hbm, outb_hbm,
               den_hbm,
               src_res, dst_res, ea_res, den_res, krows, qrows,
               vrows0, vrows1, srows0, srows1, zbuf, acc_sh,
               sg0, sg1, skq, ss0, ss1):
    c = lax.axis_index("c")
    s = lax.axis_index("s")
    wid = c * 16 + s
    nvec = _DV // 16

    lane = lax.iota(jnp.int32, 16)

    def zb(i, carry):
        r = i // nvec
        t = i % nvec
        zbuf[r, pl.ds(t * 16, 16)] = jnp.zeros((16,), jnp.float32)
        return carry
    lax.fori_loop(0, _ZR * nvec, zb, 0)

    def zd(i, carry):
        den_res[pl.ds(i * 16, 16)] = jnp.zeros((16,), jnp.float32)
        return carry
    lax.fori_loop(0, _N // 16, zd, 0)

    def zero_acc(k, carry):
        q = k * 16 + s

        @pl.when(q < _NZC)
        def _():
            pltpu.sync_copy(zbuf, acc_sh.at[pl.ds(q * _ZR, _ZR)])
        return carry

    pltpu.sync_copy(src_hbm.at[wid], src_res)
    pltpu.sync_copy(dst_hbm.at[wid], dst_res)
    pltpu.sync_copy(ea_hbm.at[wid], ea_res)

    vb = (vrows0, vrows1)
    sb = (srows0, srows1)
    sg = (sg0, sg1)
    ss = (ss0, ss1)

    for sweep in (0, 1):
        v_hbm = (va_hbm, vb_hbm)[sweep]
        out_hbm = (outa_hbm, outb_hbm)[sweep]

        lax.fori_loop(0, (_NZC + 15) // 16, zero_acc, 0)
        plsc.subcore_barrier()

        def issue_g(j, p):
            pltpu.async_copy(v_hbm.at[src_res.at[j]], vb[p], sg[p])

        def issue_kq(j):
            if sweep == 0:
                pltpu.async_copy(k_hbm.at[src_res.at[j]], krows, skq)
                pltpu.async_copy(q_hbm.at[dst_res.at[j]], qrows, skq)

        def wait_g(p):
            pltpu.make_async_copy(v_hbm.at[src_res.at[0]], vb[p],
                                  sg[p]).wait()
            if sweep == 0:
                pltpu.make_async_copy(k_hbm.at[src_res.at[0]], krows,
                                      skq).wait()
                pltpu.make_async_copy(q_hbm.at[dst_res.at[0]], qrows,
                                      skq).wait()

        def wait_s(p):
            pltpu.make_async_copy(sb[p], acc_sh.at[dst_res.at[0]],
                                  ss[p]).wait()

        def compute(j, p):
            def group(g, gcarry):
                rows = g * 16 + lane
                if sweep == 0:
                    accs = [jnp.zeros((16,), jnp.float32) for _ in range(4)]
                    for jj in range(_D):
                        col = jnp.full((16,), jj, jnp.int32)
                        kx = plsc.load_gather(krows, [rows, col])
                        qx = plsc.load_gather(qrows, [rows, col])
                        accs[jj % 4] = accs[jj % 4] + kx * qx
                    att = (accs[0] + accs[1]) + (accs[2] + accs[3])
                    ea = ea_res[j, pl.ds(g * 16, 16)]
                    e = jnp.exp(att * ea)
                    ea_res[j, pl.ds(g * 16, 16)] = e
                    dvec = dst_res[j, pl.ds(g * 16, 16)]
                    plsc.addupdate_scatter(den_res, [dvec], e)
                else:
                    e = ea_res[j, pl.ds(g * 16, 16)]
                for jj in range(_DV):
                    col = jnp.full((16,), jj, jnp.int32)
                    vv = plsc.load_gather(vb[p], [rows, col])
                    plsc.store_scatter(sb[p], [rows, col], vv * e)
                return gcarry
            lax.fori_loop(0, _C // 16, group, 0)

        def scat(j, p):
            pltpu.async_copy(sb[p], acc_sh.at[dst_res.at[j]], ss[p],
                             add=True)

        issue_kq(0)
        issue_g(0, 0)

        def pair(i, carry):
            j0 = 2 * i
            issue_g(j0 + 1, 1)
            wait_g(0)

            @pl.when(i > 0)
            def _():
                wait_s(0)
            compute(j0, 0)
            issue_kq(j0 + 1)
            scat(j0, 0)
            issue_g(j0 + 2, 0)
            wait_g(1)

            @pl.when(i > 0)
            def _():
                wait_s(1)
            compute(j0 + 1, 1)
            issue_kq(j0 + 2)
            scat(j0 + 1, 1)
            return carry
        lax.fori_loop(0, (_NCH - 1) // 2, pair, 0)

        wait_g(0)
        wait_s(0)
        compute(_NCH - 1, 0)
        scat(_NCH - 1, 0)
        wait_s(0)
        wait_s(1)

        plsc.subcore_barrier()

        def copy_out(k, carry):
            q = k * 16 + s

            @pl.when(q < _NZC)
            def _():
                pltpu.sync_copy(acc_sh.at[pl.ds(q * _ZR, _ZR)], zbuf)
                pltpu.sync_copy(zbuf, out_hbm.at[c * _NZC + q])
            return carry
        lax.fori_loop(0, (_NZC + 15) // 16, copy_out, 0)
        plsc.subcore_barrier()

        lax.fori_loop(0, _ZR * nvec, zb, 0)

    pltpu.sync_copy(den_res, den_hbm.at[wid])


def _edge_pass(kk, qq, va, vb, src3, dst3, ea3):
    mesh = plsc.VectorSubcoreMesh(core_axis_name="c", subcore_axis_name="s")
    fn = pl.kernel(
        _edge_body,
        out_type=[jax.ShapeDtypeStruct((2 * _NZC, _ZR, _DV), jnp.float32),
                  jax.ShapeDtypeStruct((2 * _NZC, _ZR, _DV), jnp.float32),
                  jax.ShapeDtypeStruct((_NT, _N), jnp.float32)],
        mesh=mesh,
        compiler_params=pltpu.CompilerParams(use_tc_tiling_on_sc=False,
                                             needs_layout_passes=False),
        scratch_types=[
            pltpu.VMEM((_NCH, _C), jnp.int32),
            pltpu.VMEM((_NCH, _C), jnp.int32),
            pltpu.VMEM((_NCH, _C), jnp.float32),
            pltpu.VMEM((_N,), jnp.float32),
            pltpu.VMEM((_C, _D), jnp.float32),
            pltpu.VMEM((_C, _D), jnp.float32),
            pltpu.VMEM((_C, _DV), jnp.float32),
            pltpu.VMEM((_C, _DV), jnp.float32),
            pltpu.VMEM((_C, _DV), jnp.float32),
            pltpu.VMEM((_C, _DV), jnp.float32),
            pltpu.VMEM((_ZR, _DV), jnp.float32),
            pltpu.VMEM_SHARED((_N, _DV), jnp.float32),
            pltpu.SemaphoreType.DMA,
            pltpu.SemaphoreType.DMA,
            pltpu.SemaphoreType.DMA,
            pltpu.SemaphoreType.DMA,
            pltpu.SemaphoreType.DMA,
        ],
    )
    return fn(kk, qq, va, vb, src3, dst3, ea3)


def _pad_w(w):
    do, di = w.shape
    return jnp.pad(w, ((0, _D - do), (0, _D - di)))


def _pad_b(b):
    return jnp.pad(b, (0, _D - b.shape[0])).reshape(1, _D)


def kernel(x, edge_index, edge_attr, batch, wk1, bk1, wq1, bq1, wv1, bv1,
           wr1, br1, wk2, bk2, wq2, bq2, wv2, bv2, wr2, br2):
    ei = edge_index.astype(jnp.int32)
    src3 = ei[0].reshape(_NT, _NCH, _C)
    dst3 = ei[1].reshape(_NT, _NCH, _C)
    ea3 = edge_attr.reshape(_NT, _NCH, _C)

    s1 = 1.0 / (64.0 ** 0.5)
    s2 = 1.0 / (128.0 ** 0.5)
    wk = jnp.stack([_pad_w(wk1 * s1), _pad_w(wk2 * s2)])
    bk = jnp.stack([_pad_b(bk1 * s1), _pad_b(bk2 * s2)])
    wq = jnp.stack([_pad_w(wq1), _pad_w(wq2)])
    bq = jnp.stack([_pad_b(bq1), _pad_b(bq2)])
    wv = jnp.stack([_pad_w(wv1), _pad_w(wv2)])
    bv = jnp.stack([_pad_b(bv1), _pad_b(bv2)])
    wr = jnp.stack([_pad_w(wr1), _pad_w(wr2)])
    br = jnp.stack([_pad_b(br1), _pad_b(br2)])
    msk = jnp.stack([jnp.ones((1, _D), jnp.float32),
                     jnp.concatenate([jnp.ones((1, 64), jnp.float32),
                                      jnp.zeros((1, 64), jnp.float32)],
                                     axis=1)])
    invd = jnp.array([[[1.0 / 128.0]], [[1.0 / 64.0]]], jnp.float32)

    per = (wk, bk, wq, bq, wv, bv, wr, br, msk, invd)

    def body(carry):
        i, h = carry
        (wk_i, bk_i, wq_i, bq_i, wv_i, bv_i, wr_i, br_i, m_i, inv_i) = (
            lax.dynamic_index_in_dim(p, i, 0, keepdims=False) for p in per)
        kqv, r = _stage_proj(h, m_i, inv_i, wk_i, bk_i, wq_i,
                             bq_i, wv_i, bv_i, wr_i, br_i)
        kk = kqv[:, :_D]
        qq = kqv[:, _D:2 * _D]
        v = kqv[:, 2 * _D:]
        va = v[:, :_DV]
        vb = v[:, _DV:]
        acc_a, acc_b, den32 = _edge_pass(
            kk, qq, va, vb, src3, dst3, ea3)
        acc_abc = jnp.concatenate([acc_a.reshape(2, _N, _DV),
                                   acc_b.reshape(2, _N, _DV)], axis=2)
        den = jnp.sum(den32.astype(jnp.float32), axis=0).reshape(_N, 1)
        h_next = _stage_post(acc_abc, den, r)
        return i + 1, h_next

    bound = 2 + jnp.minimum(batch.astype(jnp.int32)[0], 0)
    _, h2 = lax.while_loop(lambda ca: ca[0] < bound, body,
                           (jnp.int32(0), x))
    return _stage_pool(h2, batch)


# speedup vs baseline: 1.2894x; 1.2894x over previous
import jax
import jax.numpy as jnp
from jax import lax
from jax.experimental import pallas as pl
from jax.experimental.pallas import tpu as pltpu
from jax.experimental.pallas import tpu_sc as plsc

_N = 10000
_E = 320000
_G = 64
_D = 128
_DV = 48
_NSW = 3


_NT = 32
_TPT = _E // _NT
_C = 80
_NCH = _TPT // _C
_ZR = 40
_NZC = _N // _ZR

_ROWS_BLK = 400
_NBLK = _N // _ROWS_BLK


def _gelu(x):
    return 0.5 * x * (1.0 + lax.erf(x * 0.7071067811865476))


def _mm(a, b):
    return lax.dot_general(a, b, (((1,), (1,)), ((), ())),
                           preferred_element_type=jnp.float32)


def _proj_body(x_ref, m_ref, inv_ref, wk, bk, wq, bq, wv, bv, wr, br,
               kqv_out, r_out):
    xb = x_ref[...]
    m = m_ref[...]
    inv = inv_ref[0, 0]
    mu = jnp.sum(xb * m, axis=-1, keepdims=True) * inv
    xc = (xb - mu) * m
    var = jnp.sum(xc * xc, axis=-1, keepdims=True) * inv
    xn = xc * lax.rsqrt(var + 1e-5)
    k = _mm(xn, wk[...]) + bk[...]
    q = _mm(xn, wq[...]) + bq[...]
    v = _mm(xn, wv[...]) + bv[...]
    kqv_out[...] = jnp.concatenate([k, q, v], axis=1)
    r_out[...] = _mm(xb, wr[...]) + br[...]


def _stage_proj(x, msk, invd, wk, bk, wq, bq, wv, bv, wr, br):
    wspec = pl.BlockSpec((_D, _D), lambda i: (0, 0))
    bspec = pl.BlockSpec((1, _D), lambda i: (0, 0))
    return pl.pallas_call(
        _proj_body,
        grid=(_NBLK,),
        in_specs=[pl.BlockSpec((_ROWS_BLK, _D), lambda i: (i, 0)),
                  bspec, pl.BlockSpec((1, 1), lambda i: (0, 0)),
                  wspec, bspec, wspec, bspec, wspec, bspec, wspec, bspec],
        out_specs=[pl.BlockSpec((_ROWS_BLK, 3 * _D), lambda i: (i, 0)),
                   pl.BlockSpec((_ROWS_BLK, _D), lambda i: (i, 0))],
        out_shape=[jax.ShapeDtypeStruct((_N, 3 * _D), jnp.float32),
                   jax.ShapeDtypeStruct((_N, _D), jnp.float32)],
    )(x, msk, invd, wk, bk, wq, bq, wv, bv, wr, br)


def _post_body(ab_ref, den_ref, r_ref, h_out):
    s = ab_ref[0] + ab_ref[1]
    num = s[:, :_D]
    den = den_ref[...]
    aggr = num / (den + 1e-16)
    h_out[...] = _gelu(aggr) - r_ref[...]


def _stage_post(acc_abc, den, r):
    return pl.pallas_call(
        _post_body,
        grid=(_NBLK,),
        in_specs=[pl.BlockSpec((2, _ROWS_BLK, 3 * _DV), lambda i: (0, i, 0)),
                  pl.BlockSpec((_ROWS_BLK, 1), lambda i: (i, 0)),
                  pl.BlockSpec((_ROWS_BLK, _D), lambda i: (i, 0))],
        out_specs=pl.BlockSpec((_ROWS_BLK, _D), lambda i: (i, 0)),
        out_shape=jax.ShapeDtypeStruct((_N, _D), jnp.float32),
    )(acc_abc, den, r)


def _pool_body(h_ref, b_ref, pooled_ref, counts_ref):
    i = pl.program_id(0)
    h = h_ref[...]
    bvec = b_ref[0]
    oh = (lax.broadcasted_iota(jnp.int32, (_G, _ROWS_BLK), 0)
          == jnp.broadcast_to(bvec, (_G, _ROWS_BLK))).astype(jnp.float32)
    contrib = lax.dot_general(oh, h, (((1,), (0,)), ((), ())),
                              preferred_element_type=jnp.float32)
    cnt = jnp.broadcast_to(jnp.sum(oh, axis=1, keepdims=True), (_G, _D))

    @pl.when(i == 0)
    def _():
        pooled_ref[...] = contrib
        counts_ref[...] = cnt

    @pl.when(i > 0)
    def _():
        pooled_ref[...] += contrib
        counts_ref[...] += cnt

    @pl.when(i == _NBLK - 1)
    def _():
        pooled_ref[...] = pooled_ref[...] / jnp.maximum(counts_ref[...], 1.0)


def _stage_pool(h, batch):
    batch3 = batch.astype(jnp.int32).reshape(_NBLK, 1, _ROWS_BLK)
    gspec = pl.BlockSpec((_G, _D), lambda i: (0, 0))
    pooled, _ = pl.pallas_call(
        _pool_body,
        grid=(_NBLK,),
        in_specs=[pl.BlockSpec((_ROWS_BLK, _D), lambda i: (i, 0)),
                  pl.BlockSpec((1, 1, _ROWS_BLK), lambda i: (i, 0, 0))],
        out_specs=[gspec, gspec],
        out_shape=[jax.ShapeDtypeStruct((_G, _D), jnp.float32),
                   jax.ShapeDtypeStruct((_G, _D), jnp.float32)],
    )(h, batch3)
    return pooled


def _edge_body(k_hbm, q_hbm, va_hbm, vb_hbm, vc_hbm, src_hbm,
               dst_hbm, ea_hbm, outa_hbm, outb_hbm, outc_hbm,
               den_hbm,
               src_res, dst_res, ea_res, den_res, krows, qrows,
               vrows0, vrows1, srows0, srows1, zbuf, acc_sh,
               sg0, sg1, skq, ss0, ss1):
    c = lax.axis_index("c")
    s = lax.axis_index("s")
    wid = c * 16 + s
    nvec = _DV // 16

    lane = lax.iota(jnp.int32, 16)

    def zb(i, carry):
        r = i // nvec
        t = i % nvec
        zbuf[r, pl.ds(t * 16, 16)] = jnp.zeros((16,), jnp.float32)
        return carry
    lax.fori_loop(0, _ZR * nvec, zb, 0)

    def zd(i, carry):
        den_res[pl.ds(i * 16, 16)] = jnp.zeros((16,), jnp.float32)
        return carry
    lax.fori_loop(0, _N // 16, zd, 0)

    def zero_acc(k, carry):
        q = k * 16 + s

        @pl.when(q < _NZC)
        def _():
            pltpu.sync_copy(zbuf, acc_sh.at[pl.ds(q * _ZR, _ZR)])
        return carry

    pltpu.sync_copy(src_hbm.at[wid], src_res)
    pltpu.sync_copy(dst_hbm.at[wid], dst_res)
    pltpu.sync_copy(ea_hbm.at[wid], ea_res)

    vb = (vrows0, vrows1)
    sb = (srows0, srows1)
    sg = (sg0, sg1)
    ss = (ss0, ss1)

    for sweep in (0, 1, 2):
        v_hbm = (va_hbm, vb_hbm, vc_hbm)[sweep]
        out_hbm = (outa_hbm, outb_hbm, outc_hbm)[sweep]

        lax.fori_loop(0, (_NZC + 15) // 16, zero_acc, 0)
        plsc.subcore_barrier()

        def issue_g(j, p):
            pltpu.async_copy(v_hbm.at[src_res.at[j]], vb[p], sg[p])

        def issue_kq(j):
            if sweep == 0:
                pltpu.async_copy(k_hbm.at[src_res.at[j]], krows, skq)
                pltpu.async_copy(q_hbm.at[dst_res.at[j]], qrows, skq)

        def wait_g(p):
            pltpu.make_async_copy(v_hbm.at[src_res.at[0]], vb[p],
                                  sg[p]).wait()
            if sweep == 0:
                pltpu.make_async_copy(k_hbm.at[src_res.at[0]], krows,
                                      skq).wait()
                pltpu.make_async_copy(q_hbm.at[dst_res.at[0]], qrows,
                                      skq).wait()

        def wait_s(p):
            pltpu.make_async_copy(sb[p], acc_sh.at[dst_res.at[0]],
                                  ss[p]).wait()

        def compute(j, p):
            def group(g, gcarry):
                rows = g * 16 + lane
                if sweep == 0:
                    accs = [jnp.zeros((16,), jnp.float32) for _ in range(4)]
                    for jj in range(_D):
                        col = jnp.full((16,), jj, jnp.int32)
                        kx = plsc.load_gather(krows, [rows, col])
                        qx = plsc.load_gather(qrows, [rows, col])
                        accs[jj % 4] = accs[jj % 4] + kx * qx
                    att = (accs[0] + accs[1]) + (accs[2] + accs[3])
                    ea = ea_res[j, pl.ds(g * 16, 16)]
                    e = jnp.exp(att * ea)
                    ea_res[j, pl.ds(g * 16, 16)] = e
                    dvec = dst_res[j, pl.ds(g * 16, 16)]
                    plsc.addupdate_scatter(den_res, [dvec], e)
                else:
                    e = ea_res[j, pl.ds(g * 16, 16)]
                for jj in range(_DV):
                    col = jnp.full((16,), jj, jnp.int32)
                    vv = plsc.load_gather(vb[p], [rows, col])
                    plsc.store_scatter(sb[p], [rows, col], vv * e)
                return gcarry
            lax.fori_loop(0, _C // 16, group, 0)

        def scat(j, p):
            pltpu.async_copy(sb[p], acc_sh.at[dst_res.at[j]], ss[p],
                             add=True)

        issue_kq(0)
        issue_g(0, 0)

        def pair(i, carry):
            j0 = 2 * i
            issue_g(j0 + 1, 1)
            wait_g(0)

            @pl.when(i > 0)
            def _():
                wait_s(0)
            compute(j0, 0)
            issue_kq(j0 + 1)
            scat(j0, 0)
            issue_g(j0 + 2, 0)
            wait_g(1)

            @pl.when(i > 0)
            def _():
                wait_s(1)
            compute(j0 + 1, 1)
            issue_kq(j0 + 2)
            scat(j0 + 1, 1)
            return carry
        lax.fori_loop(0, (_NCH - 1) // 2, pair, 0)

        wait_g(0)
        wait_s(0)
        compute(_NCH - 1, 0)
        scat(_NCH - 1, 0)
        wait_s(0)
        wait_s(1)

        plsc.subcore_barrier()

        def copy_out(k, carry):
            q = k * 16 + s

            @pl.when(q < _NZC)
            def _():
                pltpu.sync_copy(acc_sh.at[pl.ds(q * _ZR, _ZR)], zbuf)
                pltpu.sync_copy(zbuf, out_hbm.at[c * _NZC + q])
            return carry
        lax.fori_loop(0, (_NZC + 15) // 16, copy_out, 0)
        plsc.subcore_barrier()

        lax.fori_loop(0, _ZR * nvec, zb, 0)

    pltpu.sync_copy(den_res, den_hbm.at[wid])


def _edge_pass(kk, qq, va, vb, vc, src3, dst3, ea3):
    mesh = plsc.VectorSubcoreMesh(core_axis_name="c", subcore_axis_name="s")
    fn = pl.kernel(
        _edge_body,
        out_type=[jax.ShapeDtypeStruct((2 * _NZC, _ZR, _DV), jnp.float32),
                  jax.ShapeDtypeStruct((2 * _NZC, _ZR, _DV), jnp.float32),
                  jax.ShapeDtypeStruct((2 * _NZC, _ZR, _DV), jnp.float32),
                  jax.ShapeDtypeStruct((_NT, _N), jnp.float32)],
        mesh=mesh,
        compiler_params=pltpu.CompilerParams(use_tc_tiling_on_sc=False,
                                             needs_layout_passes=False),
        scratch_types=[
            pltpu.VMEM((_NCH, _C), jnp.int32),
            pltpu.VMEM((_NCH, _C), jnp.int32),
            pltpu.VMEM((_NCH, _C), jnp.float32),
            pltpu.VMEM((_N,), jnp.float32),
            pltpu.VMEM((_C, _D), jnp.float32),
            pltpu.VMEM((_C, _D), jnp.float32),
            pltpu.VMEM((_C, _DV), jnp.float32),
            pltpu.VMEM((_C, _DV), jnp.float32),
            pltpu.VMEM((_C, _DV), jnp.float32),
            pltpu.VMEM((_C, _DV), jnp.float32),
            pltpu.VMEM((_ZR, _DV), jnp.float32),
            pltpu.VMEM_SHARED((_N, _DV), jnp.float32),
            pltpu.SemaphoreType.DMA,
            pltpu.SemaphoreType.DMA,
            pltpu.SemaphoreType.DMA,
            pltpu.SemaphoreType.DMA,
            pltpu.SemaphoreType.DMA,
        ],
    )
    return fn(kk, qq, va, vb, vc, src3, dst3, ea3)


def _pad_w(w):
    do, di = w.shape
    return jnp.pad(w, ((0, _D - do), (0, _D - di)))


def _pad_b(b):
    return jnp.pad(b, (0, _D - b.shape[0])).reshape(1, _D)


def kernel(x, edge_index, edge_attr, batch, wk1, bk1, wq1, bq1, wv1, bv1,
           wr1, br1, wk2, bk2, wq2, bq2, wv2, bv2, wr2, br2):
    ei = edge_index.astype(jnp.int32)
    src3 = ei[0].reshape(_NT, _NCH, _C)
    dst3 = ei[1].reshape(_NT, _NCH, _C)
    ea3 = edge_attr.reshape(_NT, _NCH, _C)

    s1 = 1.0 / (64.0 ** 0.5)
    s2 = 1.0 / (128.0 ** 0.5)
    wk = jnp.stack([_pad_w(wk1 * s1), _pad_w(wk2 * s2)])
    bk = jnp.stack([_pad_b(bk1 * s1), _pad_b(bk2 * s2)])
    wq = jnp.stack([_pad_w(wq1), _pad_w(wq2)])
    bq = jnp.stack([_pad_b(bq1), _pad_b(bq2)])
    wv = jnp.stack([_pad_w(wv1), _pad_w(wv2)])
    bv = jnp.stack([_pad_b(bv1), _pad_b(bv2)])
    wr = jnp.stack([_pad_w(wr1), _pad_w(wr2)])
    br = jnp.stack([_pad_b(br1), _pad_b(br2)])
    msk = jnp.stack([jnp.ones((1, _D), jnp.float32),
                     jnp.concatenate([jnp.ones((1, 64), jnp.float32),
                                      jnp.zeros((1, 64), jnp.float32)],
                                     axis=1)])
    invd = jnp.array([[[1.0 / 128.0]], [[1.0 / 64.0]]], jnp.float32)

    per = (wk, bk, wq, bq, wv, bv, wr, br, msk, invd)

    def body(carry):
        i, h = carry
        (wk_i, bk_i, wq_i, bq_i, wv_i, bv_i, wr_i, br_i, m_i, inv_i) = (
            lax.dynamic_index_in_dim(p, i, 0, keepdims=False) for p in per)
        kqv, r = _stage_proj(h, m_i, inv_i, wk_i, bk_i, wq_i,
                             bq_i, wv_i, bv_i, wr_i, br_i)
        kk = kqv[:, :_D]
        qq = kqv[:, _D:2 * _D]
        v = kqv[:, 2 * _D:]
        va = v[:, :_DV]
        vb = v[:, _DV:2 * _DV]
        vc = jnp.concatenate(
            [v[:, 2 * _DV:], jnp.zeros((_N, 3 * _DV - _D), jnp.float32)],
            axis=1)
        acc_a, acc_b, acc_c, den32 = _edge_pass(
            kk, qq, va, vb, vc, src3, dst3, ea3)
        acc_abc = jnp.concatenate([acc_a.reshape(2, _N, _DV),
                                   acc_b.reshape(2, _N, _DV),
                                   acc_c.reshape(2, _N, _DV)], axis=2)
        den = jnp.sum(den32.astype(jnp.float32), axis=0).reshape(_N, 1)
        h_next = _stage_post(acc_abc, den, r)
        return i + 1, h_next

    bound = 2 + jnp.minimum(batch.astype(jnp.int32)[0], 0)
    _, h2 = lax.while_loop(lambda ca: ca[0] < bound, body,
                           (jnp.int32(0), x))
    return _stage_pool(h2, batch)
